# Initial kernel scaffold; baseline (speedup 1.0000x reference)
import jax
import jax.numpy as jnp
from jax.experimental import pallas as pl


def kernel(inputs, w, sparsity_mask, b):
    raise NotImplementedError("write your pallas kernel here")


# speedup vs baseline: 1.0666x; 1.0666x over previous
import functools

import jax
import jax.numpy as jnp
from jax.experimental import pallas as pl
from jax.experimental.pallas import tpu as pltpu

BM = 1024
BN = 1024
BK = 1024


def _matmul_kernel(x_ref, w_ref, m_ref, b_ref, o_ref, nk: int):
    k = pl.program_id(2)
    wm = w_ref[...] * m_ref[...]
    acc = jnp.dot(x_ref[...], wm, preferred_element_type=jnp.float32)

    @pl.when(k == 0)
    def _init():
        o_ref[...] = acc + b_ref[...].astype(jnp.float32)

    @pl.when(k != 0)
    def _accum():
        o_ref[...] += acc


def kernel(inputs, w, sparsity_mask, b):
    M, K = inputs.shape
    _, N = w.shape
    x16 = inputs.astype(jnp.bfloat16)
    w16 = w.astype(jnp.bfloat16)
    m16 = sparsity_mask.astype(jnp.bfloat16)
    b2d = b.reshape(1, N)
    nk = K // BK
    grid = (M // BM, N // BN, nk)
    out = pl.pallas_call(
        functools.partial(_matmul_kernel, nk=nk),
        grid=grid,
        in_specs=[
            pl.BlockSpec((BM, BK), lambda i, j, k: (i, k)),
            pl.BlockSpec((BK, BN), lambda i, j, k: (k, j)),
            pl.BlockSpec((BK, BN), lambda i, j, k: (k, j)),
            pl.BlockSpec((1, BN), lambda i, j, k: (0, j)),
        ],
        out_specs=pl.BlockSpec((BM, BN), lambda i, j, k: (i, j)),
        out_shape=jax.ShapeDtypeStruct((M, N), jnp.float32),
        compiler_params=pltpu.CompilerParams(
            dimension_semantics=("parallel", "parallel", "arbitrary"),
        ),
    )(x16, w16, m16, b2d)
    return out


# speedup vs baseline: 1.2247x; 1.1482x over previous
import jax
import jax.numpy as jnp
from jax.experimental import pallas as pl
from jax.experimental.pallas import tpu as pltpu

BM = 2048
BN = 2048
BK = 512
PREP_BM = 256


def _prep_kernel(w_ref, m_ref, o_ref):
    o_ref[...] = (w_ref[...] * m_ref[...]).astype(jnp.bfloat16)


def _matmul_kernel(x_ref, w_ref, b_ref, o_ref):
    k = pl.program_id(2)
    acc = jnp.dot(
        x_ref[...].astype(jnp.bfloat16),
        w_ref[...],
        preferred_element_type=jnp.float32,
    )

    @pl.when(k == 0)
    def _init():
        o_ref[...] = acc + b_ref[...]

    @pl.when(k != 0)
    def _accum():
        o_ref[...] += acc


def kernel(inputs, w, sparsity_mask, b):
    M, K = inputs.shape
    _, N = w.shape
    wm16 = pl.pallas_call(
        _prep_kernel,
        grid=(K // PREP_BM,),
        in_specs=[
            pl.BlockSpec((PREP_BM, N), lambda i: (i, 0)),
            pl.BlockSpec((PREP_BM, N), lambda i: (i, 0)),
        ],
        out_specs=pl.BlockSpec((PREP_BM, N), lambda i: (i, 0)),
        out_shape=jax.ShapeDtypeStruct((K, N), jnp.bfloat16),
        compiler_params=pltpu.CompilerParams(
            dimension_semantics=("arbitrary",),
        ),
    )(w, sparsity_mask)
    b2d = b.reshape(1, N)
    grid = (M // BM, N // BN, K // BK)
    out = pl.pallas_call(
        _matmul_kernel,
        grid=grid,
        in_specs=[
            pl.BlockSpec((BM, BK), lambda i, j, k: (i, k)),
            pl.BlockSpec((BK, BN), lambda i, j, k: (k, j)),
            pl.BlockSpec((1, BN), lambda i, j, k: (0, j)),
        ],
        out_specs=pl.BlockSpec((BM, BN), lambda i, j, k: (i, j)),
        out_shape=jax.ShapeDtypeStruct((M, N), jnp.float32),
        compiler_params=pltpu.CompilerParams(
            dimension_semantics=("parallel", "parallel", "arbitrary"),
            vmem_limit_bytes=63 * 1024 * 1024,
        ),
    )(inputs, wm16, b2d)
    return out


# speedup vs baseline: 1.3688x; 1.1177x over previous
import jax
import jax.numpy as jnp
from jax.experimental import pallas as pl
from jax.experimental.pallas import tpu as pltpu

BM = 1024
BN = 2048
BK = 1024
PREP_BM = 256


def _prep_kernel(w_ref, m_ref, o_ref):
    o_ref[...] = (w_ref[...] * m_ref[...]).astype(jnp.bfloat16)


def _matmul_kernel(x_ref, w_ref, b_ref, o_ref):
    k = pl.program_id(2)
    acc = jnp.dot(
        x_ref[...].astype(jnp.bfloat16),
        w_ref[...],
        preferred_element_type=jnp.float32,
    )

    @pl.when(k == 0)
    def _init():
        o_ref[...] = acc + b_ref[...]

    @pl.when(k != 0)
    def _accum():
        o_ref[...] += acc


def kernel(inputs, w, sparsity_mask, b):
    M, K = inputs.shape
    _, N = w.shape
    wm16 = pl.pallas_call(
        _prep_kernel,
        grid=(K // PREP_BM,),
        in_specs=[
            pl.BlockSpec((PREP_BM, N), lambda i: (i, 0)),
            pl.BlockSpec((PREP_BM, N), lambda i: (i, 0)),
        ],
        out_specs=pl.BlockSpec((PREP_BM, N), lambda i: (i, 0)),
        out_shape=jax.ShapeDtypeStruct((K, N), jnp.bfloat16),
        compiler_params=pltpu.CompilerParams(
            dimension_semantics=("arbitrary",),
        ),
    )(w, sparsity_mask)
    b2d = b.reshape(1, N)
    grid = (M // BM, N // BN, K // BK)
    out = pl.pallas_call(
        _matmul_kernel,
        grid=grid,
        in_specs=[
            pl.BlockSpec((BM, BK), lambda i, j, k: (i, k)),
            pl.BlockSpec((BK, BN), lambda i, j, k: (k, j)),
            pl.BlockSpec((1, BN), lambda i, j, k: (0, j)),
        ],
        out_specs=pl.BlockSpec((BM, BN), lambda i, j, k: (i, j)),
        out_shape=jax.ShapeDtypeStruct((M, N), jnp.float32),
        compiler_params=pltpu.CompilerParams(
            dimension_semantics=("parallel", "parallel", "arbitrary"),
            vmem_limit_bytes=63 * 1024 * 1024,
        ),
    )(inputs, wm16, b2d)
    return out


# speedup vs baseline: 1.4109x; 1.0308x over previous
import jax
import jax.numpy as jnp
from jax.experimental import pallas as pl
from jax.experimental.pallas import tpu as pltpu

BM = 1024
BN = 1024
PREP_BW = 256
PREP_BX = 512


def _prep_kernel(w_ref, m_ref, x_ref, wm_ref, x16_ref):
    wm_ref[...] = (w_ref[...] * m_ref[...]).astype(jnp.bfloat16)
    x16_ref[...] = x_ref[...].astype(jnp.bfloat16)


def _matmul_kernel(x_ref, w_ref, b_ref, o_ref):
    o_ref[...] = (
        jnp.dot(x_ref[...], w_ref[...], preferred_element_type=jnp.float32)
        + b_ref[...]
    )


def kernel(inputs, w, sparsity_mask, b):
    M, K = inputs.shape
    _, N = w.shape
    wm16, x16 = pl.pallas_call(
        _prep_kernel,
        grid=(K // PREP_BW,),
        in_specs=[
            pl.BlockSpec((PREP_BW, N), lambda i: (i, 0)),
            pl.BlockSpec((PREP_BW, N), lambda i: (i, 0)),
            pl.BlockSpec((PREP_BX, K), lambda i: (i, 0)),
        ],
        out_specs=[
            pl.BlockSpec((PREP_BW, N), lambda i: (i, 0)),
            pl.BlockSpec((PREP_BX, K), lambda i: (i, 0)),
        ],
        out_shape=[
            jax.ShapeDtypeStruct((K, N), jnp.bfloat16),
            jax.ShapeDtypeStruct((M, K), jnp.bfloat16),
        ],
        compiler_params=pltpu.CompilerParams(
            dimension_semantics=("arbitrary",),
        ),
    )(w, sparsity_mask, inputs)
    b2d = b.reshape(1, N)
    grid = (M // BM, N // BN)
    out = pl.pallas_call(
        _matmul_kernel,
        grid=grid,
        in_specs=[
            pl.BlockSpec((BM, K), lambda i, j: (i, 0)),
            pl.BlockSpec((K, BN), lambda i, j: (0, j)),
            pl.BlockSpec((1, BN), lambda i, j: (0, j)),
        ],
        out_specs=pl.BlockSpec((BM, BN), lambda i, j: (i, j)),
        out_shape=jax.ShapeDtypeStruct((M, N), jnp.float32),
        compiler_params=pltpu.CompilerParams(
            dimension_semantics=("parallel", "parallel"),
            vmem_limit_bytes=63 * 1024 * 1024,
        ),
    )(x16, wm16, b2d)
    return out
